# baseline (speedup 1.0000x reference)
import jax
import jax.numpy as jnp
from jax.experimental import pallas as pl
from jax.experimental.pallas import tpu as pltpu

_H = 32
_W = 32
_HW = _H * _W
_CC = 512


def _psrw_block_kernel(cv_ref, pix_ref, pk_ref, out_ref):
    cv = cv_ref[0]
    pk = pk_ref[0]
    d2 = jax.lax.dot_general(
        pix_ref[...], pk, (((1,), (0,)), ((), ())),
    )
    nspp = pk[6:7, :]

    far = d2 > 2.0
    s_nm = jnp.sum(jnp.where(far, cv, 0.0), axis=0, keepdims=True)
    cv_mean = s_nm / nspp
    mx = jnp.max(cv, axis=0, keepdims=True)

    qual = (cv <= cv_mean) & (d2 > 0.5)
    md2 = jnp.min(jnp.where(qual, d2, 10000.0), axis=0, keepdims=True)
    width = jnp.sqrt(md2)
    thr = jnp.clip(md2, 2.0, 20.0)

    outm = d2 > thr
    s_side = jnp.sum(jnp.where(outm, cv, 0.0), axis=0, keepdims=True)
    s2_side = jnp.sum(jnp.where(outm, cv * cv, 0.0), axis=0, keepdims=True)
    nsp = jnp.sum(jnp.where(outm, 1.0, 0.0), axis=0, keepdims=True)
    mean_side = s_side / nsp
    var_side = (s2_side - s_side * mean_side) / (nsp - 1.0)

    out_ref[...] = ((mx - mean_side) / (var_side * width + 1e-16))[None]


def _norm_kernel(p_ref, out_ref):
    p = p_ref[:, 0, :]
    out_ref[...] = p / (jnp.mean(p, axis=1, keepdims=True) + 1e-8)


def kernel(cost_volume, peak_coords, mesh):
    B_, C_, H_, W_ = cost_volume.shape
    HW = H_ * W_
    cvt = jnp.transpose(cost_volume, (0, 2, 3, 1)).reshape(B_, HW, C_)

    pyi = peak_coords[..., 0]
    pxi = peak_coords[..., 1]
    pyf = pyi.astype(jnp.float32)
    pxf = pxi.astype(jnp.float32)
    n3 = ((3 - (pyi == 0) - (pyi == H_ - 1))
          * (3 - (pxi == 0) - (pxi == W_ - 1))).astype(jnp.float32)
    ones = jnp.ones_like(pyf)
    zero = jnp.zeros_like(pyf)
    wp = pyi * pyi + pxi * pxi
    wp_hi = ((wp // 32) * 32).astype(jnp.float32)
    wp_lo = (wp % 32).astype(jnp.float32)
    pk = jnp.stack(
        [-2.0 * pyf, -2.0 * pxf, ones, ones,
         wp_hi, wp_lo, float(HW) - n3, zero], axis=1)

    jj = jnp.arange(HW, dtype=jnp.int32)
    yi = jj // W_
    xi = jj % W_
    yj = yi.astype(jnp.float32)
    xj = xi.astype(jnp.float32)
    vj = yi * yi + xi * xi
    vj_hi = ((vj // 32) * 32).astype(jnp.float32)
    vj_lo = (vj % 32).astype(jnp.float32)
    onesj = jnp.ones_like(yj)
    zeroj = jnp.zeros_like(yj)
    pix = jnp.stack(
        [yj, xj, vj_hi, vj_lo, onesj, onesj, zeroj, zeroj], axis=1)

    raw = pl.pallas_call(
        _psrw_block_kernel,
        grid=(B_, C_ // _CC),
        in_specs=[
            pl.BlockSpec((1, HW, _CC), lambda b, c: (b, 0, c)),
            pl.BlockSpec((HW, 8), lambda b, c: (0, 0)),
            pl.BlockSpec((1, 8, _CC), lambda b, c: (b, 0, c)),
        ],
        out_specs=pl.BlockSpec((1, 1, _CC), lambda b, c: (b, 0, c)),
        out_shape=jax.ShapeDtypeStruct((B_, 1, C_), jnp.float32),
        compiler_params=pltpu.CompilerParams(
            dimension_semantics=("parallel", "parallel")),
    )(cvt, pix, pk)

    return pl.pallas_call(
        _norm_kernel,
        out_shape=jax.ShapeDtypeStruct((B_, C_), jnp.float32),
    )(raw)


# speedup vs baseline: 1.1552x; 1.1552x over previous
import jax
import jax.numpy as jnp
from jax.experimental import pallas as pl
from jax.experimental.pallas import tpu as pltpu

_H = 32
_W = 32
_HW = _H * _W
_CC = 512


def _psrw_block_kernel(cv_ref, pix_ref, pk_ref, out_ref):
    cv = cv_ref[0]
    pk = pk_ref[0]
    d2 = jax.lax.dot_general(
        pix_ref[...], pk, (((1,), (0,)), ((), ())),
    )
    nspp = pk[6:7, :]

    far = d2 > 2.0
    s_nm = jnp.sum(jnp.where(far, cv, 0.0), axis=0, keepdims=True)
    cv_mean = s_nm / nspp
    mx = jnp.max(cv, axis=0, keepdims=True)

    qual = (cv <= cv_mean) & (d2 > 0.5)
    md2 = jnp.min(jnp.where(qual, d2, 10000.0), axis=0, keepdims=True)
    width = jnp.sqrt(md2)
    thr = jnp.clip(md2, 2.0, 20.0)

    outm = d2 > thr
    u = jnp.where(outm, cv, 0.0)
    s_side = jnp.sum(u, axis=0, keepdims=True)
    s2_side = jnp.sum(u * u, axis=0, keepdims=True)

    py = -0.5 * pk[0:1, :]
    px = -0.5 * pk[1:2, :]
    nd = jnp.zeros_like(thr)
    for dy in range(-4, 5):
        t = thr - float(dy * dy)
        s = jnp.floor(jnp.sqrt(jnp.maximum(t, 0.0)))
        cx = jnp.minimum(s, px) + jnp.minimum(s, 31.0 - px) + 1.0
        oky = (py + float(dy) >= 0.0) & (py + float(dy) <= 31.0) & (t >= 0.0)
        nd = nd + jnp.where(oky, cx, 0.0)
    nsp = float(_HW) - nd

    mean_side = s_side / nsp
    var_side = (s2_side - s_side * mean_side) / (nsp - 1.0)

    out_ref[...] = ((mx - mean_side) / (var_side * width + 1e-16))[None]


def _norm_kernel(p_ref, out_ref):
    p = p_ref[:, 0, :]
    out_ref[...] = p / (jnp.mean(p, axis=1, keepdims=True) + 1e-8)


def kernel(cost_volume, peak_coords, mesh):
    B_, C_, H_, W_ = cost_volume.shape
    HW = H_ * W_
    cvt = jnp.transpose(cost_volume, (0, 2, 3, 1)).reshape(B_, HW, C_)

    pyi = peak_coords[..., 0]
    pxi = peak_coords[..., 1]
    pyf = pyi.astype(jnp.float32)
    pxf = pxi.astype(jnp.float32)
    n3 = ((3 - (pyi == 0) - (pyi == H_ - 1))
          * (3 - (pxi == 0) - (pxi == W_ - 1))).astype(jnp.float32)
    ones = jnp.ones_like(pyf)
    zero = jnp.zeros_like(pyf)
    wp = pyi * pyi + pxi * pxi
    wp_hi = ((wp // 32) * 32).astype(jnp.float32)
    wp_lo = (wp % 32).astype(jnp.float32)
    pk = jnp.stack(
        [-2.0 * pyf, -2.0 * pxf, ones, ones,
         wp_hi, wp_lo, float(HW) - n3, zero], axis=1)

    jj = jnp.arange(HW, dtype=jnp.int32)
    yi = jj // W_
    xi = jj % W_
    yj = yi.astype(jnp.float32)
    xj = xi.astype(jnp.float32)
    vj = yi * yi + xi * xi
    vj_hi = ((vj // 32) * 32).astype(jnp.float32)
    vj_lo = (vj % 32).astype(jnp.float32)
    onesj = jnp.ones_like(yj)
    zeroj = jnp.zeros_like(yj)
    pix = jnp.stack(
        [yj, xj, vj_hi, vj_lo, onesj, onesj, zeroj, zeroj], axis=1)

    raw = pl.pallas_call(
        _psrw_block_kernel,
        grid=(B_, C_ // _CC),
        in_specs=[
            pl.BlockSpec((1, HW, _CC), lambda b, c: (b, 0, c)),
            pl.BlockSpec((HW, 8), lambda b, c: (0, 0)),
            pl.BlockSpec((1, 8, _CC), lambda b, c: (b, 0, c)),
        ],
        out_specs=pl.BlockSpec((1, 1, _CC), lambda b, c: (b, 0, c)),
        out_shape=jax.ShapeDtypeStruct((B_, 1, C_), jnp.float32),
        compiler_params=pltpu.CompilerParams(
            dimension_semantics=("parallel", "parallel")),
    )(cvt, pix, pk)

    return pl.pallas_call(
        _norm_kernel,
        out_shape=jax.ShapeDtypeStruct((B_, C_), jnp.float32),
    )(raw)


# speedup vs baseline: 1.2356x; 1.0695x over previous
import jax
import jax.numpy as jnp
from jax.experimental import pallas as pl
from jax.experimental.pallas import tpu as pltpu

_H = 32
_W = 32
_HW = _H * _W
_CC = 1024


def _psrw_block_kernel(cv_ref, pix_ref, pk_ref, out_ref):
    cv = cv_ref[0]
    pk = pk_ref[0]
    d2 = jax.lax.dot_general(
        pix_ref[...], pk, (((1,), (0,)), ((), ())),
    )
    nspp = pk[6:7, :]

    far = d2 > 2.0
    s_nm = jnp.sum(jnp.where(far, cv, 0.0), axis=0, keepdims=True)
    cv_mean = s_nm / nspp
    mx = jnp.max(cv, axis=0, keepdims=True)

    qual = (cv <= cv_mean) & (d2 > 0.5)
    md2 = jnp.min(jnp.where(qual, d2, 10000.0), axis=0, keepdims=True)
    width = jnp.sqrt(md2)
    thr = jnp.clip(md2, 2.0, 20.0)

    outm = d2 > thr
    u = jnp.where(outm, cv, 0.0)
    s_side = jnp.sum(u, axis=0, keepdims=True)
    s2_side = jnp.sum(u * u, axis=0, keepdims=True)

    py = -0.5 * pk[0:1, :]
    px = -0.5 * pk[1:2, :]
    nd = jnp.zeros_like(thr)
    for dy in range(-4, 5):
        t = thr - float(dy * dy)
        s = jnp.floor(jnp.sqrt(jnp.maximum(t, 0.0)))
        cx = jnp.minimum(s, px) + jnp.minimum(s, 31.0 - px) + 1.0
        oky = (py + float(dy) >= 0.0) & (py + float(dy) <= 31.0) & (t >= 0.0)
        nd = nd + jnp.where(oky, cx, 0.0)
    nsp = float(_HW) - nd

    mean_side = s_side / nsp
    var_side = (s2_side - s_side * mean_side) / (nsp - 1.0)

    psrw = (mx - mean_side) / (var_side * width + 1e-16)
    out_ref[...] = (psrw / (jnp.mean(psrw, axis=1, keepdims=True) + 1e-8))[None]


def kernel(cost_volume, peak_coords, mesh):
    B_, C_, H_, W_ = cost_volume.shape
    HW = H_ * W_
    cvt = jnp.transpose(cost_volume, (0, 2, 3, 1)).reshape(B_, HW, C_)

    pyi = peak_coords[..., 0]
    pxi = peak_coords[..., 1]
    pyf = pyi.astype(jnp.float32)
    pxf = pxi.astype(jnp.float32)
    n3 = ((3 - (pyi == 0) - (pyi == H_ - 1))
          * (3 - (pxi == 0) - (pxi == W_ - 1))).astype(jnp.float32)
    ones = jnp.ones_like(pyf)
    zero = jnp.zeros_like(pyf)
    wp = pyi * pyi + pxi * pxi
    wp_hi = ((wp // 32) * 32).astype(jnp.float32)
    wp_lo = (wp % 32).astype(jnp.float32)
    pk = jnp.stack(
        [-2.0 * pyf, -2.0 * pxf, ones, ones,
         wp_hi, wp_lo, float(HW) - n3, zero], axis=1)

    jj = jnp.arange(HW, dtype=jnp.int32)
    yi = jj // W_
    xi = jj % W_
    yj = yi.astype(jnp.float32)
    xj = xi.astype(jnp.float32)
    vj = yi * yi + xi * xi
    vj_hi = ((vj // 32) * 32).astype(jnp.float32)
    vj_lo = (vj % 32).astype(jnp.float32)
    onesj = jnp.ones_like(yj)
    zeroj = jnp.zeros_like(yj)
    pix = jnp.stack(
        [yj, xj, vj_hi, vj_lo, onesj, onesj, zeroj, zeroj], axis=1)

    raw = pl.pallas_call(
        _psrw_block_kernel,
        grid=(B_,),
        in_specs=[
            pl.BlockSpec((1, HW, _CC), lambda b: (b, 0, 0)),
            pl.BlockSpec((HW, 8), lambda b: (0, 0)),
            pl.BlockSpec((1, 8, _CC), lambda b: (b, 0, 0)),
        ],
        out_specs=pl.BlockSpec((1, 1, _CC), lambda b: (b, 0, 0)),
        out_shape=jax.ShapeDtypeStruct((B_, 1, C_), jnp.float32),
        compiler_params=pltpu.CompilerParams(
            dimension_semantics=("parallel",)),
    )(cvt, pix, pk)

    return raw.reshape(B_, C_)


# speedup vs baseline: 1.3318x; 1.0779x over previous
import math

import numpy as np

import jax
import jax.numpy as jnp
from jax.experimental import pallas as pl
from jax.experimental.pallas import tpu as pltpu

_H = 32
_W = 32
_HW = _H * _W


def _pix_table() -> np.ndarray:
    jj = np.arange(_HW)
    yj = (jj // _W).astype(np.float64)
    xj = (jj % _W).astype(np.float64)
    vj = yj * yj + xj * xj
    vj_hi = np.floor(vj / 32.0) * 32.0
    vj_lo = vj - vj_hi
    ones = np.ones_like(yj)
    return np.stack(
        [-2.0 * yj, -2.0 * xj, vj_hi, vj_lo, ones, ones],
        axis=1).astype(np.float32)


_PIX = _pix_table()


def _psrw_kernel(cv_ref, pix_ref, pc_ref, out_ref):
    cv = cv_ref[0]
    pcf = pc_ref[0].astype(jnp.float32)
    py = pcf[0:1, :]
    px = pcf[1:2, :]
    wp = py * py + px * px
    wp_hi = jnp.floor(wp * (1.0 / 32.0)) * 32.0
    wp_lo = wp - wp_hi
    onesr = jnp.ones_like(py)
    rhs = jnp.concatenate([py, px, onesr, onesr, wp_hi, wp_lo], axis=0)

    d2 = jax.lax.dot_general(
        pix_ref[...], rhs, (((1,), (0,)), ((), ())))

    n3 = ((3.0 - (py == 0.0) - (py == 31.0))
          * (3.0 - (px == 0.0) - (px == 31.0)))
    nspp = float(_HW) - n3

    far = d2 > 2.0
    s_nm = jnp.sum(jnp.where(far, cv, 0.0), axis=0, keepdims=True)
    cv_mean = s_nm / nspp
    mx = jnp.max(cv, axis=0, keepdims=True)

    qual = (cv <= cv_mean) & (d2 > 0.5)
    md2 = jnp.min(jnp.where(qual, d2, 10000.0), axis=0, keepdims=True)
    width = jnp.sqrt(md2)
    thr = jnp.clip(md2, 2.0, 20.0)

    outm = d2 > thr
    u = jnp.where(outm, cv, 0.0)
    s_side = jnp.sum(u, axis=0, keepdims=True)
    s2_side = jnp.sum(u * u, axis=0, keepdims=True)

    nd = jnp.zeros_like(thr)
    for dy in range(-4, 5):
        t = thr - float(dy * dy)
        s = jnp.floor(jnp.sqrt(jnp.maximum(t, 0.0)))
        cx = jnp.minimum(s, px) + jnp.minimum(s, 31.0 - px) + 1.0
        oky = (py + float(dy) >= 0.0) & (py + float(dy) <= 31.0) & (t >= 0.0)
        nd = nd + jnp.where(oky, cx, 0.0)
    nsp = float(_HW) - nd

    mean_side = s_side / nsp
    var_side = (s2_side - s_side * mean_side) / (nsp - 1.0)

    psrw = (mx - mean_side) / (var_side * width + 1e-16)
    out_ref[...] = (psrw / (jnp.mean(psrw, axis=1, keepdims=True) + 1e-8))[None]


def kernel(cost_volume, peak_coords, mesh):
    B_, C_, H_, W_ = cost_volume.shape
    HW = H_ * W_
    cvt = jnp.transpose(cost_volume, (0, 2, 3, 1)).reshape(B_, HW, C_)
    pct = jnp.transpose(peak_coords, (0, 2, 1))

    raw = pl.pallas_call(
        _psrw_kernel,
        grid=(B_,),
        in_specs=[
            pl.BlockSpec((1, HW, C_), lambda b: (b, 0, 0)),
            pl.BlockSpec((HW, 6), lambda b: (0, 0)),
            pl.BlockSpec((1, 2, C_), lambda b: (b, 0, 0)),
        ],
        out_specs=pl.BlockSpec((1, 1, C_), lambda b: (b, 0, 0)),
        out_shape=jax.ShapeDtypeStruct((B_, 1, C_), jnp.float32),
        compiler_params=pltpu.CompilerParams(
            dimension_semantics=("parallel",)),
    )(cvt, jnp.asarray(_PIX), pct)

    return raw.reshape(B_, C_)
